# Initial kernel scaffold; baseline (speedup 1.0000x reference)
import jax
import jax.numpy as jnp
from jax.experimental import pallas as pl


def kernel(features, edge_index, W_self0, W_neigh0, b0, W_self1, W_neigh1, b1):
    raise NotImplementedError("write your pallas kernel here")


# speedup vs baseline: 4.5334x; 4.5334x over previous
import functools

import jax
import jax.numpy as jnp
from jax import lax
from jax.experimental import pallas as pl
from jax.experimental.pallas import tpu as pltpu
from jax.experimental.pallas import tpu_sc as plsc

_NC = 2
_NS = 16
_NW = _NC * _NS
_C = 128


def _segment_partials(table, src_r, dst_r, n_pad):
    n_chunks = src_r.shape[1]
    W = table.shape[1]
    rps = n_pad // _NS
    nz = rps // _C

    mesh = plsc.VectorSubcoreMesh(core_axis_name="c", subcore_axis_name="s")

    @functools.partial(
        pl.kernel,
        mesh=mesh,
        compiler_params=pltpu.CompilerParams(use_tc_tiling_on_sc=False),
        out_type=jax.ShapeDtypeStruct((2 * n_pad, W), jnp.float32),
        scratch_types=[
            pltpu.VMEM((n_chunks, _C), jnp.int32),
            pltpu.VMEM((n_chunks, _C), jnp.int32),
            pltpu.VMEM((_C, W), jnp.float32),
            pltpu.VMEM_SHARED((n_pad, W), jnp.float32),
            pltpu.SemaphoreType.DMA,
        ],
    )
    def sc_kernel(table_hbm, src_hbm, dst_hbm, out_hbm,
                  src_v, dst_v, buf, acc, sem):
        cid = lax.axis_index("c")
        sid = lax.axis_index("s")
        wid = sid * _NC + cid

        zvec = jnp.zeros((16,), jnp.float32)

        def _zrow(i, carry):
            for j in range(W // 16):
                buf[i, pl.ds(j * 16, 16)] = zvec
            return carry

        lax.fori_loop(0, _C, _zrow, 0)

        def _zcopy(t, carry):
            pltpu.sync_copy(buf, acc.at[pl.ds(sid * rps + t * _C, _C)])
            return carry

        lax.fori_loop(0, nz, _zcopy, 0)

        pltpu.sync_copy(src_hbm.at[wid], src_v)
        pltpu.sync_copy(dst_hbm.at[wid], dst_v)
        plsc.subcore_barrier()

        def _edge_chunk(j, carry):
            pltpu.async_copy(table_hbm.at[src_v.at[j]], buf, sem).wait()
            pltpu.sync_copy(buf, acc.at[dst_v.at[j]], add=True)
            return carry

        lax.fori_loop(0, n_chunks, _edge_chunk, 0)

        plsc.subcore_barrier()
        base = sid * rps
        pltpu.sync_copy(acc.at[pl.ds(base, rps)],
                        out_hbm.at[pl.ds(cid * n_pad + base, rps)])

    return sc_kernel(table, src_r, dst_r)


_BLK = 1000


def _mm(a, b):
    return jnp.dot(a, b, precision=jax.lax.Precision.HIGHEST)


def _layer_mid(x, pa, pb, da, db, W_self0, W_neigh0, b0, W_neigh1):
    n, d = x.shape
    grid = (n // _BLK,)

    def body(x_ref, pa_ref, pb_ref, da_ref, db_ref,
             ws_ref, wn_ref, b_ref, wn1_ref, h_ref, p1_ref):
        deg = jnp.sum(da_ref[...] + db_ref[...], axis=1, keepdims=True) * (1.0 / 16.0)
        inv = 1.0 / jnp.maximum(deg, 1.0)
        agg = (pa_ref[...] + pb_ref[...]) * inv
        h = jnp.maximum(_mm(x_ref[...], ws_ref[...]) + _mm(agg, wn_ref[...])
                        + b_ref[...], 0.0)
        h_ref[...] = h
        p1_ref[...] = _mm(h, wn1_ref[...])

    row = pl.BlockSpec((_BLK, d), lambda i: (i, 0))
    row16 = pl.BlockSpec((_BLK, 16), lambda i: (i, 0))
    full = pl.BlockSpec((d, d), lambda i: (0, 0))
    vec = pl.BlockSpec((1, d), lambda i: (0, 0))
    return pl.pallas_call(
        body,
        grid=grid,
        in_specs=[row, row, row, row16, row16, full, full, vec, full],
        out_specs=[row, row],
        out_shape=[jax.ShapeDtypeStruct((n, d), jnp.float32),
                   jax.ShapeDtypeStruct((n, d), jnp.float32)],
    )(x, pa, pb, da, db, W_self0, W_neigh0, b0[None, :], W_neigh1)


def _layer_out(h, qa, qb, da, db, W_self1, b1):
    n, d = h.shape
    grid = (n // _BLK,)

    def body(h_ref, qa_ref, qb_ref, da_ref, db_ref, ws_ref, b_ref, o_ref):
        deg = jnp.sum(da_ref[...] + db_ref[...], axis=1, keepdims=True) * (1.0 / 16.0)
        inv = 1.0 / jnp.maximum(deg, 1.0)
        o_ref[...] = (_mm(h_ref[...], ws_ref[...])
                      + (qa_ref[...] + qb_ref[...]) * inv + b_ref[...])

    row = pl.BlockSpec((_BLK, d), lambda i: (i, 0))
    row16 = pl.BlockSpec((_BLK, 16), lambda i: (i, 0))
    full = pl.BlockSpec((d, d), lambda i: (0, 0))
    vec = pl.BlockSpec((1, d), lambda i: (0, 0))
    return pl.pallas_call(
        body,
        grid=grid,
        in_specs=[row, row, row, row16, row16, full, vec],
        out_specs=row,
        out_shape=jax.ShapeDtypeStruct((n, d), jnp.float32),
    )(h, qa, qb, da, db, W_self1, b1[None, :])


def kernel(features, edge_index, W_self0, W_neigh0, b0, W_self1, W_neigh1, b1):
    n, d = features.shape
    e = edge_index.shape[1]
    src = edge_index[0]
    dst = edge_index[1]

    n_chunks = -(-e // (_NW * _C))
    e_pad = _NW * n_chunks * _C
    n_pad = -(-n // (_NS * _C)) * (_NS * _C)

    src_r = jnp.concatenate(
        [src, jnp.zeros((e_pad - e,), jnp.int32)]).reshape(_NW, n_chunks, _C)
    dst_r = jnp.concatenate(
        [dst, jnp.full((e_pad - e,), n, jnp.int32)]).reshape(_NW, n_chunks, _C)

    aug = jnp.concatenate([features, jnp.ones((n, 16), jnp.float32)], axis=1)
    part0 = _segment_partials(aug, src_r, dst_r, n_pad)
    pa, da = part0[:n, :d], part0[:n, d:]
    pb, db = part0[n_pad:n_pad + n, :d], part0[n_pad:n_pad + n, d:]

    h, p1 = _layer_mid(features, pa, pb, da, db, W_self0, W_neigh0, b0, W_neigh1)

    part1 = _segment_partials(p1, src_r, dst_r, n_pad)
    qa = part1[:n]
    qb = part1[n_pad:n_pad + n]

    return _layer_out(h, qa, qb, da, db, W_self1, b1)
